# baseline (speedup 1.0000x reference)
import functools

import jax
import jax.numpy as jnp
from jax import lax
from jax.experimental import pallas as pl
from jax.experimental.pallas import tpu as pltpu
from jax.experimental.pallas import tpu_sc as plsc

_N = 10000
_E = 160000
_DF = 128
_DE = 16
_H = 8
_HP = 16
_OUT = 3

_NC = 2
_NS = 16
_NW = _NC * _NS

_CH = 128
_NCHUNK = _E // _CH
_CPW = -(-_NCHUNK // _NW)
_LASTW = _NCHUNK - (_NW - 1) * _CPW
_NCHUNK_PAD = _CPW * _NW
_RPS = 624
_RPS_LAST = _N - (_NS - 1) * _RPS

def _worker_id():
    return lax.axis_index("s") * _NC + lax.axis_index("c")


@functools.cache
def _sc_kernels():
    mesh = plsc.VectorSubcoreMesh(
        core_axis_name="c", subcore_axis_name="s",
        num_cores=_NC, num_subcores=_NS,
    )
    linear = pltpu.CompilerParams(use_tc_tiling_on_sc=False)

    @functools.partial(
        pl.kernel,
        out_type=jax.ShapeDtypeStruct((_E, _DF), jnp.bfloat16),
        mesh=mesh,
        scratch_types=[
            pltpu.VMEM((_CPW, _CH), jnp.int32),
            pltpu.VMEM((_CH, _DF), jnp.bfloat16),
            pltpu.VMEM((_CH, _DF), jnp.bfloat16),
            pltpu.SemaphoreType.DMA,
            pltpu.SemaphoreType.DMA,
        ],
        compiler_params=linear,
    )
    def gather_x(x_hbm, src2d_hbm, out_hbm, idx_v, rows0, rows1, sem0, sem1):
        w = _worker_id()
        cbase = w * _CPW
        pltpu.sync_copy(src2d_hbm.at[pl.ds(cbase, _CPW)], idx_v)
        nchunks = jnp.where(w == _NW - 1, _LASTW, _CPW)

        def body(p, carry):
            j0 = 2 * p
            j1 = j0 + 1
            d0 = pltpu.async_copy(x_hbm.at[idx_v.at[j0]], rows0, sem0)
            d1 = pltpu.async_copy(x_hbm.at[idx_v.at[j1]], rows1, sem1)
            d0.wait()
            pltpu.sync_copy(rows0, out_hbm.at[pl.ds((cbase + j0) * _CH, _CH)])
            d1.wait()
            pltpu.sync_copy(rows1, out_hbm.at[pl.ds((cbase + j1) * _CH, _CH)])
            return carry

        lax.fori_loop(0, nchunks // 2, body, 0)

    @functools.partial(
        pl.kernel,
        out_type=jax.ShapeDtypeStruct((_NC, _N, _HP), jnp.float32),
        mesh=mesh,
        scratch_types=[
            pltpu.VMEM((_CPW, _CH), jnp.int32),
            pltpu.VMEM((_CH, _HP), jnp.float32),
            pltpu.VMEM((_CH, _HP), jnp.float32),
            pltpu.VMEM_SHARED((_N, _HP), jnp.float32),
            pltpu.SemaphoreType.DMA,
            pltpu.SemaphoreType.DMA,
        ],
        compiler_params=linear,
    )
    def scatter_msg(dst2d_hbm, msg_hbm, zeros_hbm, out_hbm,
                    idx_v, rows0, rows1, acc_sh, sem0, sem1):
        c = lax.axis_index("c")
        s = lax.axis_index("s")
        w = _worker_id()

        @pl.when(s == 0)
        def _():
            pltpu.sync_copy(zeros_hbm, acc_sh)

        plsc.subcore_barrier()

        pltpu.sync_copy(dst2d_hbm.at[pl.ds(w * _CPW, _CPW)], idx_v)
        nchunks = jnp.where(w == _NW - 1, _LASTW, _CPW)

        def body(p, carry):
            j0 = 2 * p
            j1 = j0 + 1
            base0 = (w * _CPW + j0) * _CH
            base1 = (w * _CPW + j1) * _CH
            d0 = pltpu.async_copy(msg_hbm.at[pl.ds(base0, _CH)], rows0, sem0)
            d1 = pltpu.async_copy(msg_hbm.at[pl.ds(base1, _CH)], rows1, sem1)
            d0.wait()
            pltpu.sync_copy(rows0, acc_sh.at[idx_v.at[j0]], add=True)
            d1.wait()
            pltpu.sync_copy(rows1, acc_sh.at[idx_v.at[j1]], add=True)
            return carry

        lax.fori_loop(0, nchunks // 2, body, 0)

        plsc.subcore_barrier()

        @pl.when(s < _NS - 1)
        def _():
            pltpu.sync_copy(acc_sh.at[pl.ds(s * _RPS, _RPS)],
                            out_hbm.at[c, pl.ds(s * _RPS, _RPS)])

        @pl.when(s == _NS - 1)
        def _():
            pltpu.sync_copy(acc_sh.at[pl.ds((_NS - 1) * _RPS, _RPS_LAST)],
                            out_hbm.at[c, pl.ds((_NS - 1) * _RPS, _RPS_LAST)])

    @functools.partial(
        pl.kernel,
        out_type=(
            jax.ShapeDtypeStruct((_E, _HP), jnp.float32),
            jax.ShapeDtypeStruct((_E, _HP), jnp.float32),
        ),
        mesh=mesh,
        scratch_types=[
            pltpu.VMEM((_CPW, _CH), jnp.int32),
            pltpu.VMEM((_CPW, _CH), jnp.int32),
            pltpu.VMEM((_CH, _HP), jnp.float32),
            pltpu.VMEM((_CH, _HP), jnp.float32),
            pltpu.VMEM((_CH, _HP), jnp.float32),
            pltpu.VMEM((_CH, _HP), jnp.float32),
            pltpu.VMEM_SHARED((_N, _HP), jnp.float32),
            pltpu.SemaphoreType.DMA,
            pltpu.SemaphoreType.DMA,
            pltpu.SemaphoreType.DMA,
            pltpu.SemaphoreType.DMA,
        ],
        compiler_params=linear,
    )
    def gather_h(h_hbm, src2d_hbm, dst2d_hbm, hs_hbm, hd_hbm,
                 idxs_v, idxd_v, rs0, rd0, rs1, rd1, h_sh,
                 sa, sb, sc, sd):
        s = lax.axis_index("s")
        w = _worker_id()
        cbase = w * _CPW

        @pl.when(s == 0)
        def _():
            pltpu.sync_copy(h_hbm, h_sh)

        pltpu.sync_copy(src2d_hbm.at[pl.ds(cbase, _CPW)], idxs_v)
        pltpu.sync_copy(dst2d_hbm.at[pl.ds(cbase, _CPW)], idxd_v)
        plsc.subcore_barrier()
        nchunks = jnp.where(w == _NW - 1, _LASTW, _CPW)

        def body(p, carry):
            j0 = 2 * p
            j1 = j0 + 1
            base0 = (cbase + j0) * _CH
            base1 = (cbase + j1) * _CH
            ds0 = pltpu.async_copy(h_sh.at[idxs_v.at[j0]], rs0, sa)
            dd0 = pltpu.async_copy(h_sh.at[idxd_v.at[j0]], rd0, sb)
            ds1 = pltpu.async_copy(h_sh.at[idxs_v.at[j1]], rs1, sc)
            dd1 = pltpu.async_copy(h_sh.at[idxd_v.at[j1]], rd1, sd)
            ds0.wait()
            pltpu.sync_copy(rs0, hs_hbm.at[pl.ds(base0, _CH)])
            dd0.wait()
            pltpu.sync_copy(rd0, hd_hbm.at[pl.ds(base0, _CH)])
            ds1.wait()
            pltpu.sync_copy(rs1, hs_hbm.at[pl.ds(base1, _CH)])
            dd1.wait()
            pltpu.sync_copy(rd1, hd_hbm.at[pl.ds(base1, _CH)])
            return carry

        lax.fori_loop(0, nchunks // 2, body, 0)

    return gather_x, scatter_msg, gather_h


_BE = 2000
_BE2 = 2000


def _msg_body(ea_ref, h_ref, We1_ref, be1_ref, We2P_ref, be2P_ref, S_ref, out_ref):
    a = jnp.maximum(
        jnp.dot(ea_ref[...], We1_ref[...], preferred_element_type=jnp.float32)
        + be1_ref[...], 0.0)
    weP = jnp.dot(a, We2P_ref[...],
                  preferred_element_type=jnp.float32) + be2P_ref[...]
    h8 = jnp.concatenate([h_ref[...].astype(jnp.float32)] * _H, axis=1)
    out_ref[...] = jnp.dot(weP * h8, S_ref[...],
                           preferred_element_type=jnp.float32)


def _msg_call(ea, h_src, We1, be1, We2P, be2P, S):
    grid = (_E // _BE,)
    return pl.pallas_call(
        _msg_body,
        grid=grid,
        in_specs=[
            pl.BlockSpec((_BE, _DE), lambda i: (i, 0)),
            pl.BlockSpec((_BE, _DF), lambda i: (i, 0)),
            pl.BlockSpec((_DE, _H * _DF), lambda i: (0, 0)),
            pl.BlockSpec((1, _H * _DF), lambda i: (0, 0)),
            pl.BlockSpec((_H * _DF, _H * _DF), lambda i: (0, 0)),
            pl.BlockSpec((1, _H * _DF), lambda i: (0, 0)),
            pl.BlockSpec((_H * _DF, _HP), lambda i: (0, 0)),
        ],
        out_specs=pl.BlockSpec((_BE, _HP), lambda i: (i, 0)),
        out_shape=jax.ShapeDtypeStruct((_E, _HP), jnp.float32),
    )(ea, h_src, We1, be1, We2P, be2P, S)


def _combine_body(p_ref, b_ref, h_ref):
    h_ref[...] = jnp.maximum(p_ref[0] + p_ref[1] + b_ref[...], 0.0)


def _combine_call(partials, b16):
    return pl.pallas_call(
        _combine_body,
        out_shape=jax.ShapeDtypeStruct((_N, _HP), jnp.float32),
    )(partials, b16)


def _cls_body(hs_ref, hd_ref, ea_ref, W1s_ref, W1d_ref, W1e_ref, b1_ref,
              W2_ref, b2_ref, out_ref):
    z = (jnp.dot(hs_ref[...], W1s_ref[...], preferred_element_type=jnp.float32)
         + jnp.dot(hd_ref[...], W1d_ref[...], preferred_element_type=jnp.float32)
         + jnp.dot(ea_ref[...], W1e_ref[...], preferred_element_type=jnp.float32)
         + b1_ref[...])
    z = jnp.maximum(z, 0.0)
    out_ref[...] = (jnp.dot(z, W2_ref[...], preferred_element_type=jnp.float32)
                    + b2_ref[...])


def _cls_call(hs2, hd2, ea2, W1s_bd, W1d_bd, W1e_bd, b1_bd, W2_bd, b2_bd):
    rows = _E // 8
    grid = (rows // _BE2,)
    return pl.pallas_call(
        _cls_body,
        grid=grid,
        in_specs=[
            pl.BlockSpec((_BE2, 128), lambda i: (i, 0)),
            pl.BlockSpec((_BE2, 128), lambda i: (i, 0)),
            pl.BlockSpec((_BE2, 128), lambda i: (i, 0)),
            pl.BlockSpec((128, 64), lambda i: (0, 0)),
            pl.BlockSpec((128, 64), lambda i: (0, 0)),
            pl.BlockSpec((128, 64), lambda i: (0, 0)),
            pl.BlockSpec((1, 64), lambda i: (0, 0)),
            pl.BlockSpec((64, 128), lambda i: (0, 0)),
            pl.BlockSpec((1, 128), lambda i: (0, 0)),
        ],
        out_specs=pl.BlockSpec((_BE2, 128), lambda i: (i, 0)),
        out_shape=jax.ShapeDtypeStruct((rows, 128), jnp.float32),
    )(hs2, hd2, ea2, W1s_bd, W1d_bd, W1e_bd, b1_bd, W2_bd, b2_bd)


def kernel(x, edge_index, edge_attr, edge_indices, W_e1, b_e1, W_e2, b_e2,
           b_conv, W_c1, b_c1, W_c2, b_c2):
    del edge_indices
    src = edge_index[0]
    dst = edge_index[1]
    pad = _NCHUNK_PAD * _CH - _E
    src2d = jnp.concatenate([src, jnp.zeros((pad,), jnp.int32)]).reshape(
        _NCHUNK_PAD, _CH)
    dst2d = jnp.concatenate([dst, jnp.zeros((pad,), jnp.int32)]).reshape(
        _NCHUNK_PAD, _CH)

    K = _H * _DF
    We2P = W_e2.reshape(K, _DF, _H).transpose(0, 2, 1).reshape(K, K)
    be2P = b_e2.reshape(_DF, _H).T.reshape(1, K)
    S = (jnp.arange(K, dtype=jnp.int32)[:, None] // _DF
         == jnp.arange(_HP, dtype=jnp.int32)[None, :]).astype(jnp.float32)

    gather_x, scatter_msg, gather_h = _sc_kernels()

    h_src = gather_x(x.astype(jnp.bfloat16), src2d)
    msg = _msg_call(edge_attr, h_src, W_e1, b_e1.reshape(1, K), We2P, be2P, S)

    partials = scatter_msg(dst2d, msg, jnp.zeros((_N, _HP), jnp.float32))
    b16 = jnp.concatenate([b_conv, jnp.zeros((_HP - _H,), jnp.float32)]).reshape(1, _HP)
    h16 = _combine_call(partials, b16)

    hs, hd = gather_h(h16, src2d, dst2d)

    W1s = jnp.zeros((_HP, _H), jnp.float32).at[:_H].set(W_c1[:_H])
    W1d = jnp.zeros((_HP, _H), jnp.float32).at[:_H].set(W_c1[_H:2 * _H])
    W1e = jnp.zeros((_HP, _H), jnp.float32).at[:_DE].set(W_c1[2 * _H:])
    W2p = jnp.zeros((_H, _HP), jnp.float32).at[:, :_OUT].set(W_c2)
    b2p = jnp.zeros((_HP,), jnp.float32).at[:_OUT].set(b_c2)
    eye8 = jnp.eye(8, dtype=jnp.float32)
    ea2 = edge_attr.reshape(_E // 8, 128)
    logits = _cls_call(
        hs.reshape(_E // 8, 128), hd.reshape(_E // 8, 128), ea2,
        jnp.kron(eye8, W1s), jnp.kron(eye8, W1d), jnp.kron(eye8, W1e),
        jnp.tile(b_c1, 8).reshape(1, 64),
        jnp.kron(eye8, W2p), jnp.tile(b2p, 8).reshape(1, 128))
    return logits.reshape(_E, _HP)[:, :_OUT]


# speedup vs baseline: 1.1782x; 1.1782x over previous
import functools

import jax
import jax.numpy as jnp
from jax import lax
from jax.experimental import pallas as pl
from jax.experimental.pallas import tpu as pltpu
from jax.experimental.pallas import tpu_sc as plsc

_N = 10000
_E = 160000
_DF = 128
_DE = 16
_H = 8
_HP = 16
_OUT = 3

_NC = 2
_NS = 16
_NW = _NC * _NS

_CH = 128
_NCHUNK = _E // _CH
_CPW = -(-_NCHUNK // _NW)
_LASTW = _NCHUNK - (_NW - 1) * _CPW
_NCHUNK_PAD = _CPW * _NW
_RPS = 624
_RPS_LAST = _N - (_NS - 1) * _RPS

def _worker_id():
    return lax.axis_index("s") * _NC + lax.axis_index("c")


@functools.cache
def _sc_kernels():
    mesh = plsc.VectorSubcoreMesh(
        core_axis_name="c", subcore_axis_name="s",
        num_cores=_NC, num_subcores=_NS,
    )
    linear = pltpu.CompilerParams(use_tc_tiling_on_sc=False)

    @functools.partial(
        pl.kernel,
        out_type=jax.ShapeDtypeStruct((_E, _DF), jnp.float32),
        mesh=mesh,
        scratch_types=[
            pltpu.VMEM((_CPW, _CH), jnp.int32),
            pltpu.VMEM((_CH, _DF), jnp.float32),
            pltpu.VMEM((_CH, _DF), jnp.float32),
            pltpu.SemaphoreType.DMA,
            pltpu.SemaphoreType.DMA,
        ],
    )
    def gather_x(x_hbm, src2d_hbm, out_hbm, idx_v, rows0, rows1, sem0, sem1):
        w = _worker_id()
        cbase = w * _CPW
        pltpu.sync_copy(src2d_hbm.at[pl.ds(cbase, _CPW)], idx_v)
        nchunks = jnp.where(w == _NW - 1, _LASTW, _CPW)

        def body(p, carry):
            j0 = 2 * p
            j1 = j0 + 1
            d0 = pltpu.async_copy(x_hbm.at[idx_v.at[j0]], rows0, sem0)
            d1 = pltpu.async_copy(x_hbm.at[idx_v.at[j1]], rows1, sem1)
            d0.wait()
            pltpu.sync_copy(rows0, out_hbm.at[pl.ds((cbase + j0) * _CH, _CH)])
            d1.wait()
            pltpu.sync_copy(rows1, out_hbm.at[pl.ds((cbase + j1) * _CH, _CH)])
            return carry

        lax.fori_loop(0, nchunks // 2, body, 0)

    @functools.partial(
        pl.kernel,
        out_type=jax.ShapeDtypeStruct((_NC, _N, _HP), jnp.float32),
        mesh=mesh,
        scratch_types=[
            pltpu.VMEM((_CPW, _CH), jnp.int32),
            pltpu.VMEM((_CH, _HP), jnp.float32),
            pltpu.VMEM((_CH, _HP), jnp.float32),
            pltpu.VMEM_SHARED((_N, _HP), jnp.float32),
            pltpu.SemaphoreType.DMA,
            pltpu.SemaphoreType.DMA,
            pltpu.SemaphoreType.DMA,
            pltpu.SemaphoreType.DMA,
        ],
        compiler_params=linear,
    )
    def scatter_msg(dst2d_hbm, msg_hbm, zeros_hbm, out_hbm,
                    idx_v, rows0, rows1, acc_sh, sem0, sem1, sem2, sem3):
        c = lax.axis_index("c")
        s = lax.axis_index("s")
        w = _worker_id()

        @pl.when(s == 0)
        def _():
            pltpu.sync_copy(zeros_hbm, acc_sh)

        plsc.subcore_barrier()

        pltpu.sync_copy(dst2d_hbm.at[pl.ds(w * _CPW, _CPW)], idx_v)
        nchunks = jnp.where(w == _NW - 1, _LASTW, _CPW)

        def body(p, carry):
            j0 = 2 * p
            j1 = j0 + 1
            base0 = (w * _CPW + j0) * _CH
            base1 = (w * _CPW + j1) * _CH
            d0 = pltpu.async_copy(msg_hbm.at[pl.ds(base0, _CH)], rows0, sem0)
            d1 = pltpu.async_copy(msg_hbm.at[pl.ds(base1, _CH)], rows1, sem1)
            d0.wait()
            a0 = pltpu.async_copy(rows0, acc_sh.at[idx_v.at[j0]], sem2, add=True)
            d1.wait()
            a1 = pltpu.async_copy(rows1, acc_sh.at[idx_v.at[j1]], sem3, add=True)
            a0.wait()
            a1.wait()
            return carry

        lax.fori_loop(0, nchunks // 2, body, 0)

        plsc.subcore_barrier()

        @pl.when(s < _NS - 1)
        def _():
            pltpu.sync_copy(acc_sh.at[pl.ds(s * _RPS, _RPS)],
                            out_hbm.at[c, pl.ds(s * _RPS, _RPS)])

        @pl.when(s == _NS - 1)
        def _():
            pltpu.sync_copy(acc_sh.at[pl.ds((_NS - 1) * _RPS, _RPS_LAST)],
                            out_hbm.at[c, pl.ds((_NS - 1) * _RPS, _RPS_LAST)])

    @functools.partial(
        pl.kernel,
        out_type=(
            jax.ShapeDtypeStruct((_E, _HP), jnp.float32),
            jax.ShapeDtypeStruct((_E, _HP), jnp.float32),
        ),
        mesh=mesh,
        scratch_types=[
            pltpu.VMEM((_CPW, _CH), jnp.int32),
            pltpu.VMEM((_CPW, _CH), jnp.int32),
            pltpu.VMEM((_CH, _HP), jnp.float32),
            pltpu.VMEM((_CH, _HP), jnp.float32),
            pltpu.VMEM((_CH, _HP), jnp.float32),
            pltpu.VMEM((_CH, _HP), jnp.float32),
            pltpu.VMEM_SHARED((_N, _HP), jnp.float32),
            pltpu.SemaphoreType.DMA,
            pltpu.SemaphoreType.DMA,
            pltpu.SemaphoreType.DMA,
            pltpu.SemaphoreType.DMA,
        ],
        compiler_params=linear,
    )
    def gather_h(h_hbm, src2d_hbm, dst2d_hbm, hs_hbm, hd_hbm,
                 idxs_v, idxd_v, rs0, rd0, rs1, rd1, h_sh,
                 sa, sb, sc, sd):
        s = lax.axis_index("s")
        w = _worker_id()
        cbase = w * _CPW

        @pl.when(s == 0)
        def _():
            pltpu.sync_copy(h_hbm, h_sh)

        pltpu.sync_copy(src2d_hbm.at[pl.ds(cbase, _CPW)], idxs_v)
        pltpu.sync_copy(dst2d_hbm.at[pl.ds(cbase, _CPW)], idxd_v)
        plsc.subcore_barrier()
        nchunks = jnp.where(w == _NW - 1, _LASTW, _CPW)

        def body(p, carry):
            j0 = 2 * p
            j1 = j0 + 1
            base0 = (cbase + j0) * _CH
            base1 = (cbase + j1) * _CH
            ds0 = pltpu.async_copy(h_sh.at[idxs_v.at[j0]], rs0, sa)
            dd0 = pltpu.async_copy(h_sh.at[idxd_v.at[j0]], rd0, sb)
            ds1 = pltpu.async_copy(h_sh.at[idxs_v.at[j1]], rs1, sc)
            dd1 = pltpu.async_copy(h_sh.at[idxd_v.at[j1]], rd1, sd)
            ds0.wait()
            pltpu.sync_copy(rs0, hs_hbm.at[pl.ds(base0, _CH)])
            dd0.wait()
            pltpu.sync_copy(rd0, hd_hbm.at[pl.ds(base0, _CH)])
            ds1.wait()
            pltpu.sync_copy(rs1, hs_hbm.at[pl.ds(base1, _CH)])
            dd1.wait()
            pltpu.sync_copy(rd1, hd_hbm.at[pl.ds(base1, _CH)])
            return carry

        lax.fori_loop(0, nchunks // 2, body, 0)

    return gather_x, scatter_msg, gather_h


_BE = 2000
_BE2 = 2000


def _msg_body(ea_ref, h_ref, We1_ref, be1_ref, We2P_ref, be2P_ref, S_ref, out_ref):
    a = jnp.maximum(
        jnp.dot(ea_ref[...], We1_ref[...], preferred_element_type=jnp.float32)
        + be1_ref[...], 0.0)
    weP = jnp.dot(a, We2P_ref[...],
                  preferred_element_type=jnp.float32) + be2P_ref[...]
    h8 = jnp.concatenate([h_ref[...]] * _H, axis=1)
    out_ref[...] = jnp.dot(weP * h8, S_ref[...],
                           preferred_element_type=jnp.float32)


def _msg_call(ea, h_src, We1, be1, We2P, be2P, S):
    grid = (_E // _BE,)
    return pl.pallas_call(
        _msg_body,
        grid=grid,
        in_specs=[
            pl.BlockSpec((_BE, _DE), lambda i: (i, 0)),
            pl.BlockSpec((_BE, _DF), lambda i: (i, 0)),
            pl.BlockSpec((_DE, _H * _DF), lambda i: (0, 0)),
            pl.BlockSpec((1, _H * _DF), lambda i: (0, 0)),
            pl.BlockSpec((_H * _DF, _H * _DF), lambda i: (0, 0)),
            pl.BlockSpec((1, _H * _DF), lambda i: (0, 0)),
            pl.BlockSpec((_H * _DF, _HP), lambda i: (0, 0)),
        ],
        out_specs=pl.BlockSpec((_BE, _HP), lambda i: (i, 0)),
        out_shape=jax.ShapeDtypeStruct((_E, _HP), jnp.float32),
    )(ea, h_src, We1, be1, We2P, be2P, S)


def _combine_body(p_ref, b_ref, h_ref):
    h_ref[...] = jnp.maximum(p_ref[0] + p_ref[1] + b_ref[...], 0.0)


def _combine_call(partials, b16):
    return pl.pallas_call(
        _combine_body,
        out_shape=jax.ShapeDtypeStruct((_N, _HP), jnp.float32),
    )(partials, b16)


def _cls_body(hs_ref, hd_ref, ea_ref, W1s_ref, W1d_ref, W1e_ref, b1_ref,
              W2_ref, b2_ref, out_ref):
    z = (jnp.dot(hs_ref[...], W1s_ref[...], preferred_element_type=jnp.float32)
         + jnp.dot(hd_ref[...], W1d_ref[...], preferred_element_type=jnp.float32)
         + jnp.dot(ea_ref[...], W1e_ref[...], preferred_element_type=jnp.float32)
         + b1_ref[...])
    z = jnp.maximum(z, 0.0)
    out_ref[...] = (jnp.dot(z, W2_ref[...], preferred_element_type=jnp.float32)
                    + b2_ref[...])


def _cls_call(hs2, hd2, ea2, W1s_bd, W1d_bd, W1e_bd, b1_bd, W2_bd, b2_bd):
    rows = _E // 8
    grid = (rows // _BE2,)
    return pl.pallas_call(
        _cls_body,
        grid=grid,
        in_specs=[
            pl.BlockSpec((_BE2, 128), lambda i: (i, 0)),
            pl.BlockSpec((_BE2, 128), lambda i: (i, 0)),
            pl.BlockSpec((_BE2, 128), lambda i: (i, 0)),
            pl.BlockSpec((128, 64), lambda i: (0, 0)),
            pl.BlockSpec((128, 64), lambda i: (0, 0)),
            pl.BlockSpec((128, 64), lambda i: (0, 0)),
            pl.BlockSpec((1, 64), lambda i: (0, 0)),
            pl.BlockSpec((64, 128), lambda i: (0, 0)),
            pl.BlockSpec((1, 128), lambda i: (0, 0)),
        ],
        out_specs=pl.BlockSpec((_BE2, 128), lambda i: (i, 0)),
        out_shape=jax.ShapeDtypeStruct((rows, 128), jnp.float32),
    )(hs2, hd2, ea2, W1s_bd, W1d_bd, W1e_bd, b1_bd, W2_bd, b2_bd)


def kernel(x, edge_index, edge_attr, edge_indices, W_e1, b_e1, W_e2, b_e2,
           b_conv, W_c1, b_c1, W_c2, b_c2):
    del edge_indices
    src = edge_index[0]
    dst = edge_index[1]
    pad = _NCHUNK_PAD * _CH - _E
    src2d = jnp.concatenate([src, jnp.zeros((pad,), jnp.int32)]).reshape(
        _NCHUNK_PAD, _CH)
    dst2d = jnp.concatenate([dst, jnp.zeros((pad,), jnp.int32)]).reshape(
        _NCHUNK_PAD, _CH)

    K = _H * _DF
    We2P = W_e2.reshape(K, _DF, _H).transpose(0, 2, 1).reshape(K, K)
    be2P = b_e2.reshape(_DF, _H).T.reshape(1, K)
    S = (jnp.arange(K, dtype=jnp.int32)[:, None] // _DF
         == jnp.arange(_HP, dtype=jnp.int32)[None, :]).astype(jnp.float32)

    gather_x, scatter_msg, gather_h = _sc_kernels()

    h_src = gather_x(x, src2d)
    msg = _msg_call(edge_attr, h_src, W_e1, b_e1.reshape(1, K), We2P, be2P, S)

    partials = scatter_msg(dst2d, msg, jnp.zeros((_N, _HP), jnp.float32))
    b16 = jnp.concatenate([b_conv, jnp.zeros((_HP - _H,), jnp.float32)]).reshape(1, _HP)
    h16 = _combine_call(partials, b16)

    hs, hd = gather_h(h16, src2d, dst2d)

    W1s = jnp.zeros((_HP, _H), jnp.float32).at[:_H].set(W_c1[:_H])
    W1d = jnp.zeros((_HP, _H), jnp.float32).at[:_H].set(W_c1[_H:2 * _H])
    W1e = jnp.zeros((_HP, _H), jnp.float32).at[:_DE].set(W_c1[2 * _H:])
    W2p = jnp.zeros((_H, _HP), jnp.float32).at[:, :_OUT].set(W_c2)
    b2p = jnp.zeros((_HP,), jnp.float32).at[:_OUT].set(b_c2)
    eye8 = jnp.eye(8, dtype=jnp.float32)
    ea2 = edge_attr.reshape(_E // 8, 128)
    logits = _cls_call(
        hs.reshape(_E // 8, 128), hd.reshape(_E // 8, 128), ea2,
        jnp.kron(eye8, W1s), jnp.kron(eye8, W1d), jnp.kron(eye8, W1e),
        jnp.tile(b_c1, 8).reshape(1, 64),
        jnp.kron(eye8, W2p), jnp.tile(b2p, 8).reshape(1, 128))
    return logits.reshape(_E, _HP)[:, :_OUT]
